# Initial kernel scaffold; baseline (speedup 1.0000x reference)
import jax
import jax.numpy as jnp
from jax.experimental import pallas as pl


def kernel(inputs, embedding):
    raise NotImplementedError("write your pallas kernel here")


# speedup vs baseline: 1.0142x; 1.0142x over previous
import jax
import jax.numpy as jnp
from jax.experimental import pallas as pl

_TOK = 1024


def _vq_body(x_ref, sx_ref, esq_ref, e_ref, et_ref,
             qst_ref, idx_ref, counts_ref, loss_ref):
    i = pl.program_id(0)
    K = e_ref.shape[1]
    x = x_ref[...]
    xc = jnp.clip(x, -10.0, 10.0)
    mm = jnp.dot(xc, e_ref[...], preferred_element_type=jnp.float32)
    d = (sx_ref[...] + esq_ref[...]) - 2.0 * mm
    m = jnp.min(d, axis=1, keepdims=True)
    iota = jax.lax.broadcasted_iota(jnp.int32, d.shape, 1)
    idx = jnp.min(jnp.where(d == m, iota, K), axis=1, keepdims=True)
    idx_ref[...] = idx
    onehot = (iota == idx).astype(jnp.float32)
    q = jnp.dot(onehot, et_ref[...], preferred_element_type=jnp.float32)
    qst_ref[...] = x + (q - x)
    cnt = jnp.sum(onehot, axis=0, keepdims=True)
    lsum = jnp.sum(jnp.sum((q - x) ** 2, axis=1, keepdims=True),
                   axis=0, keepdims=True)

    @pl.when(i == 0)
    def _():
        counts_ref[...] = cnt
        loss_ref[...] = lsum

    @pl.when(i != 0)
    def _():
        counts_ref[...] += cnt
        loss_ref[...] += lsum


def kernel(inputs, embedding):
    B, D, H, W = inputs.shape
    K = embedding.shape[1]
    N = B * H * W

    x_perm = jnp.transpose(inputs.astype(jnp.float32), (0, 2, 3, 1))
    flat = x_perm.reshape(N, D)
    flat_c = jnp.clip(flat, -10.0, 10.0)
    s_x = jnp.sum(flat_c ** 2, axis=1, keepdims=True)
    e_sq = jnp.sum(embedding ** 2, axis=0, keepdims=True)
    e_t = embedding.T

    grid = N // _TOK
    qst, idx, counts, loss_sum = pl.pallas_call(
        _vq_body,
        grid=(grid,),
        in_specs=[
            pl.BlockSpec((_TOK, D), lambda i: (i, 0)),
            pl.BlockSpec((_TOK, 1), lambda i: (i, 0)),
            pl.BlockSpec((1, K), lambda i: (0, 0)),
            pl.BlockSpec((D, K), lambda i: (0, 0)),
            pl.BlockSpec((K, D), lambda i: (0, 0)),
        ],
        out_specs=[
            pl.BlockSpec((_TOK, D), lambda i: (i, 0)),
            pl.BlockSpec((_TOK, 1), lambda i: (i, 0)),
            pl.BlockSpec((1, K), lambda i: (0, 0)),
            pl.BlockSpec((1, 1), lambda i: (0, 0)),
        ],
        out_shape=[
            jax.ShapeDtypeStruct((N, D), jnp.float32),
            jax.ShapeDtypeStruct((N, 1), jnp.int32),
            jax.ShapeDtypeStruct((1, K), jnp.float32),
            jax.ShapeDtypeStruct((1, 1), jnp.float32),
        ],
    )(flat, s_x, e_sq, embedding, e_t)

    quantized_st = jnp.transpose(qst.reshape(B, H, W, D), (0, 3, 1, 2))
    quantized_st = quantized_st.astype(inputs.dtype)
    encoding_indices = idx.reshape(N)
    mean_sq = loss_sum[0, 0] / jnp.float32(N * D)
    loss = mean_sq + 0.25 * mean_sq
    avg_probs = counts[0] / jnp.float32(N)
    perplexity = jnp.exp(-jnp.sum(avg_probs * jnp.log(avg_probs + 1e-10)))
    return (quantized_st, loss, perplexity, encoding_indices)
